# Initial kernel scaffold; baseline (speedup 1.0000x reference)
import jax
import jax.numpy as jnp
from jax.experimental import pallas as pl


def kernel(inputs, Wg, w1, w2, w3):
    raise NotImplementedError("write your pallas kernel here")


# speedup vs baseline: 1.3322x; 1.3322x over previous
import functools

import jax
import jax.numpy as jnp
from jax.experimental import pallas as pl
from jax.experimental.pallas import tpu as pltpu

E = 8
TOP_K = 2
DIM = 1024
DFF = 4096
T = 32
CHUNK = 512
NJ = DFF // CHUNK


def _moe_body(x_ref, wg_ref, w1_ref, w3_ref, w2_ref, out_ref, coef_ref):
    i = pl.program_id(0)
    j = pl.program_id(1)

    @pl.when((i == 0) & (j == 0))
    def _gate():
        x = x_ref[...]
        logits = jax.lax.dot_general(
            x, wg_ref[...], (((1,), (1,)), ((), ())),
            preferred_element_type=jnp.float32)
        iota = jax.lax.broadcasted_iota(jnp.int32, (T, E), 1)
        m1 = jnp.max(logits, axis=1, keepdims=True)
        i1 = jnp.min(jnp.where(logits == m1, iota, E), axis=1, keepdims=True)
        masked = jnp.where(iota == i1, -jnp.inf, logits)
        m2 = jnp.max(masked, axis=1, keepdims=True)
        i2 = jnp.min(jnp.where(masked == m2, iota, E), axis=1, keepdims=True)
        e2 = jnp.exp(m2 - m1)
        denom = 1.0 + e2
        wa = 1.0 / denom
        wb = e2 / denom
        coef = (jnp.where(iota == i1, wa, 0.0)
                + jnp.where(iota == i2, wb, 0.0))
        coef_ref[...] = coef
        out_ref[...] = jnp.zeros_like(out_ref)

    x = x_ref[...]
    a = jax.lax.dot_general(
        x, w1_ref[0], (((1,), (1,)), ((), ())),
        preferred_element_type=jnp.float32)
    b = jax.lax.dot_general(
        x, w3_ref[0], (((1,), (1,)), ((), ())),
        preferred_element_type=jnp.float32)
    h = a * jax.lax.logistic(a) * b
    iota = jax.lax.broadcasted_iota(jnp.int32, (T, E), 1)
    c = jnp.sum(jnp.where(iota == i, coef_ref[...], 0.0), axis=1,
                keepdims=True)
    h = h * c
    out_ref[...] += jax.lax.dot_general(
        h, w2_ref[0], (((1,), (1,)), ((), ())),
        preferred_element_type=jnp.float32)


@functools.partial(jax.jit, static_argnums=())
def kernel(inputs, Wg, w1, w2, w3):
    x = inputs.reshape(-1, inputs.shape[-1])
    out = pl.pallas_call(
        _moe_body,
        grid=(E, NJ),
        in_specs=[
            pl.BlockSpec((T, DIM), lambda i, j: (0, 0)),
            pl.BlockSpec((E, DIM), lambda i, j: (0, 0)),
            pl.BlockSpec((1, CHUNK, DIM), lambda i, j: (i, j, 0)),
            pl.BlockSpec((1, CHUNK, DIM), lambda i, j: (i, j, 0)),
            pl.BlockSpec((1, DIM, CHUNK), lambda i, j: (i, 0, j)),
        ],
        out_specs=pl.BlockSpec((T, DIM), lambda i, j: (0, 0)),
        out_shape=jax.ShapeDtypeStruct((T, DIM), jnp.float32),
        scratch_shapes=[pltpu.VMEM((T, E), jnp.float32)],
    )(x, Wg, w1, w3, w2)
    return out.reshape(inputs.shape)


# speedup vs baseline: 1.3935x; 1.0460x over previous
import functools

import jax
import jax.numpy as jnp
from jax.experimental import pallas as pl
from jax.experimental.pallas import tpu as pltpu

E = 8
TOP_K = 2
DIM = 1024
DFF = 4096
T = 32
CHUNK = 1024
NJ = DFF // CHUNK


def _moe_body(x_ref, wg_ref, w1_ref, w3_ref, w2_ref, out_ref, coef_ref):
    i = pl.program_id(0)
    j = pl.program_id(1)

    @pl.when((i == 0) & (j == 0))
    def _gate():
        x = x_ref[...]
        logits = jax.lax.dot_general(
            x, wg_ref[...], (((1,), (1,)), ((), ())),
            preferred_element_type=jnp.float32)
        iota = jax.lax.broadcasted_iota(jnp.int32, (T, E), 1)
        m1 = jnp.max(logits, axis=1, keepdims=True)
        i1 = jnp.min(jnp.where(logits == m1, iota, E), axis=1, keepdims=True)
        masked = jnp.where(iota == i1, -jnp.inf, logits)
        m2 = jnp.max(masked, axis=1, keepdims=True)
        i2 = jnp.min(jnp.where(masked == m2, iota, E), axis=1, keepdims=True)
        e2 = jnp.exp(m2 - m1)
        denom = 1.0 + e2
        wa = 1.0 / denom
        wb = e2 / denom
        coef = (jnp.where(iota == i1, wa, 0.0)
                + jnp.where(iota == i2, wb, 0.0))
        coef_ref[...] = coef
        out_ref[...] = jnp.zeros_like(out_ref)

    x = x_ref[...]
    a = jax.lax.dot_general(
        x, w1_ref[0], (((1,), (1,)), ((), ())),
        preferred_element_type=jnp.float32)
    b = jax.lax.dot_general(
        x, w3_ref[0], (((1,), (1,)), ((), ())),
        preferred_element_type=jnp.float32)
    h = a * jax.lax.logistic(a) * b
    iota = jax.lax.broadcasted_iota(jnp.int32, (T, E), 1)
    c = jnp.sum(jnp.where(iota == i, coef_ref[...], 0.0), axis=1,
                keepdims=True)
    h = h * c
    out_ref[...] += jax.lax.dot_general(
        h, w2_ref[0], (((1,), (1,)), ((), ())),
        preferred_element_type=jnp.float32)


@functools.partial(jax.jit, static_argnums=())
def kernel(inputs, Wg, w1, w2, w3):
    x = inputs.reshape(-1, inputs.shape[-1])
    out = pl.pallas_call(
        _moe_body,
        grid=(E, NJ),
        in_specs=[
            pl.BlockSpec((T, DIM), lambda i, j: (0, 0)),
            pl.BlockSpec((E, DIM), lambda i, j: (0, 0)),
            pl.BlockSpec((1, CHUNK, DIM), lambda i, j: (i, j, 0)),
            pl.BlockSpec((1, CHUNK, DIM), lambda i, j: (i, j, 0)),
            pl.BlockSpec((1, DIM, CHUNK), lambda i, j: (i, 0, j)),
        ],
        out_specs=pl.BlockSpec((T, DIM), lambda i, j: (0, 0)),
        out_shape=jax.ShapeDtypeStruct((T, DIM), jnp.float32),
        scratch_shapes=[pltpu.VMEM((T, E), jnp.float32)],
    )(x, Wg, w1, w3, w2)
    return out.reshape(inputs.shape)


# speedup vs baseline: 1.4262x; 1.0235x over previous
import functools

import jax
import jax.numpy as jnp
from jax.experimental import pallas as pl
from jax.experimental.pallas import tpu as pltpu

E = 8
TOP_K = 2
DIM = 1024
DFF = 4096
T = 32
CHUNK = 2048
NJ = DFF // CHUNK


def _moe_body(x_ref, wg_ref, w1_ref, w3_ref, w2_ref, out_ref, coef_ref):
    i = pl.program_id(0)
    j = pl.program_id(1)

    @pl.when((i == 0) & (j == 0))
    def _gate():
        x = x_ref[...]
        logits = jax.lax.dot_general(
            x, wg_ref[...], (((1,), (1,)), ((), ())),
            preferred_element_type=jnp.float32)
        iota = jax.lax.broadcasted_iota(jnp.int32, (T, E), 1)
        m1 = jnp.max(logits, axis=1, keepdims=True)
        i1 = jnp.min(jnp.where(logits == m1, iota, E), axis=1, keepdims=True)
        masked = jnp.where(iota == i1, -jnp.inf, logits)
        m2 = jnp.max(masked, axis=1, keepdims=True)
        i2 = jnp.min(jnp.where(masked == m2, iota, E), axis=1, keepdims=True)
        e2 = jnp.exp(m2 - m1)
        denom = 1.0 + e2
        wa = 1.0 / denom
        wb = e2 / denom
        coef = (jnp.where(iota == i1, wa, 0.0)
                + jnp.where(iota == i2, wb, 0.0))
        coef_ref[...] = coef
        out_ref[...] = jnp.zeros_like(out_ref)

    x = x_ref[...].astype(jnp.bfloat16)
    a = jax.lax.dot_general(
        x, w1_ref[0].astype(jnp.bfloat16), (((1,), (1,)), ((), ())),
        preferred_element_type=jnp.float32)
    b = jax.lax.dot_general(
        x, w3_ref[0].astype(jnp.bfloat16), (((1,), (1,)), ((), ())),
        preferred_element_type=jnp.float32)
    h = a * jax.lax.logistic(a) * b
    iota = jax.lax.broadcasted_iota(jnp.int32, (T, E), 1)
    c = jnp.sum(jnp.where(iota == i, coef_ref[...], 0.0), axis=1,
                keepdims=True)
    h = (h * c).astype(jnp.bfloat16)
    out_ref[...] += jax.lax.dot_general(
        h, w2_ref[0].astype(jnp.bfloat16), (((1,), (1,)), ((), ())),
        preferred_element_type=jnp.float32)


@functools.partial(jax.jit, static_argnums=())
def kernel(inputs, Wg, w1, w2, w3):
    x = inputs.reshape(-1, inputs.shape[-1])
    out = pl.pallas_call(
        _moe_body,
        grid=(E, NJ),
        in_specs=[
            pl.BlockSpec((T, DIM), lambda i, j: (0, 0)),
            pl.BlockSpec((E, DIM), lambda i, j: (0, 0)),
            pl.BlockSpec((1, CHUNK, DIM), lambda i, j: (i, j, 0)),
            pl.BlockSpec((1, CHUNK, DIM), lambda i, j: (i, j, 0)),
            pl.BlockSpec((1, DIM, CHUNK), lambda i, j: (i, 0, j)),
        ],
        out_specs=pl.BlockSpec((T, DIM), lambda i, j: (0, 0)),
        out_shape=jax.ShapeDtypeStruct((T, DIM), jnp.float32),
        scratch_shapes=[pltpu.VMEM((T, E), jnp.float32)],
    )(x, Wg, w1, w3, w2)
    return out.reshape(inputs.shape)
